# baseline (speedup 1.0000x reference)
import jax
import jax.numpy as jnp
from jax import lax
from jax.experimental import pallas as pl
from jax.experimental.pallas import tpu as pltpu
from jax.experimental.pallas import tpu_sc as plsc

VOCAB = 100000
EMB = 16
BATCH = 16384
NC = 2
NS = 16
NW = NC * NS
BPW = BATCH // NW
LANES = 16
NCHUNK = BPW // LANES

LIN_ROWS = VOCAB // 8
LCH = 200

OFF_W1 = 0
OFF_B1 = 1024
OFF_W2 = 1056
OFF_B2 = 1088
WPACK = 1104

_SC_PARAMS = pltpu.CompilerParams(
    needs_layout_passes=False, use_tc_tiling_on_sc=False)
_MESH = dict(core_axis_name="c", subcore_axis_name="s",
             num_cores=NC, num_subcores=NS)


def _mlp_body(du_hbm, di_hbm, lin_u, lin_i, wp_hbm, out_hbm,
              idu, idi, rowu, rowi, offu, offi, buf, hbuf, wv, logits_v, sem):
    c = lax.axis_index("c")
    s = lax.axis_index("s")
    wid = s * NC + c

    pltpu.sync_copy(wp_hbm, wv)
    pltpu.sync_copy(du_hbm.at[pl.ds(pl.multiple_of(wid * BPW, 8), BPW)], idu)
    pltpu.sync_copy(di_hbm.at[pl.ds(pl.multiple_of(wid * BPW, 8), BPW)], idi)

    for j in range(4):
        for o in range(8):
            vu = idu[pl.ds(j * 128 + o * 16, 16)]
            vi = idi[pl.ds(j * 128 + o * 16, 16)]
            rowu[j, pl.ds(o * 16, 16)] = vu >> 3
            rowi[j, pl.ds(o * 16, 16)] = vi >> 3
            offu[pl.ds(j * 128 + o * 16, 16)] = (vu & 7) * 16
            offi[pl.ds(j * 128 + o * 16, 16)] = (vi & 7) * 16

    riota = lax.iota(jnp.int32, LANES)

    def gather(lin, row_ref):
        cps = [pltpu.async_copy(lin.at[row_ref.at[j]],
                                buf.at[pl.ds(j * 128, 128)], sem)
               for j in range(4)]
        for cp in cps:
            cp.wait()

    gather(lin_u, rowu)

    def pass1(ci, carry):
        rows = ci * LANES + riota
        off = offu[pl.ds(ci * LANES, LANES)]
        feats = [plsc.load_gather(buf, [rows, off + k]) for k in range(EMB)]
        b1a = wv[pl.ds(OFF_B1, LANES)]
        b1b = wv[pl.ds(OFF_B1 + LANES, LANES)]
        for jf in range(32):
            wa = wv[pl.ds(OFF_W1 + jf * 32, LANES)]
            bj = b1a[jf] if jf < LANES else b1b[jf - LANES]
            h = jnp.full((LANES,), 0.0, jnp.float32) + bj
            for k in range(EMB):
                h = h + feats[k] * wa[k]
            hbuf[jf, pl.ds(ci * LANES, LANES)] = h
        return carry

    lax.fori_loop(0, NCHUNK, pass1, 0)

    gather(lin_i, rowi)

    def pass2(ci, carry):
        rows = ci * LANES + riota
        off = offi[pl.ds(ci * LANES, LANES)]
        feats = [plsc.load_gather(buf, [rows, off + k]) for k in range(EMB)]
        w2a = wv[pl.ds(OFF_W2, LANES)]
        w2b = wv[pl.ds(OFF_W2 + LANES, LANES)]
        b2v = wv[pl.ds(OFF_B2, LANES)]
        acc2 = jnp.full((LANES,), 0.0, jnp.float32)
        for jf in range(32):
            wb = wv[pl.ds(OFF_W1 + jf * 32 + LANES, LANES)]
            h = hbuf[jf, pl.ds(ci * LANES, LANES)]
            for k in range(EMB):
                h = h + feats[k] * wb[k]
            h = jnp.maximum(h, 0.0)
            w2j = w2a[jf] if jf < LANES else w2b[jf - LANES]
            acc2 = acc2 + h * w2j
        logits_v[pl.ds(ci * LANES, LANES)] = acc2 + b2v[0]
        return carry

    lax.fori_loop(0, NCHUNK, pass2, 0)
    pltpu.sync_copy(
        logits_v, out_hbm.at[pl.ds(pl.multiple_of(wid * BPW, 8), BPW)])


@jax.jit
def _run(du, di, ut, it, wpack):
    lin_u = ut.reshape(LIN_ROWS, 128)
    lin_i = it.reshape(LIN_ROWS, 128)
    mlp_f = pl.kernel(
        _mlp_body,
        out_type=jax.ShapeDtypeStruct((BATCH,), jnp.float32),
        mesh=plsc.VectorSubcoreMesh(**_MESH),
        compiler_params=_SC_PARAMS,
        scratch_types=[
            pltpu.VMEM((BPW,), jnp.int32),
            pltpu.VMEM((BPW,), jnp.int32),
            pltpu.VMEM((4, 128), jnp.int32),
            pltpu.VMEM((4, 128), jnp.int32),
            pltpu.VMEM((BPW,), jnp.int32),
            pltpu.VMEM((BPW,), jnp.int32),
            pltpu.VMEM((BPW, 128), jnp.float32),
            pltpu.VMEM((32, BPW), jnp.float32),
            pltpu.VMEM((WPACK,), jnp.float32),
            pltpu.VMEM((BPW,), jnp.float32),
            pltpu.SemaphoreType.DMA,
        ],
    )
    return mlp_f(du, di, lin_u, lin_i, wpack)


def kernel(dataUser, dataItem, user_table, item_table, W1, b1, W2, b2):
    du = dataUser.astype(jnp.int32)
    di = dataItem.astype(jnp.int32)
    wpack = jnp.concatenate([
        W1.reshape(-1), b1.reshape(-1), W2.reshape(-1), b2.reshape(-1),
        jnp.zeros((WPACK - (OFF_B2 + 1),), jnp.float32)])
    out = _run(du, di, user_table, item_table, wpack)
    return out.reshape(BATCH, 1)


# speedup vs baseline: 1.1905x; 1.1905x over previous
import jax
import jax.numpy as jnp
from jax import lax
from jax.experimental import pallas as pl
from jax.experimental.pallas import tpu as pltpu
from jax.experimental.pallas import tpu_sc as plsc

VOCAB = 100000
EMB = 16
BATCH = 16384
NC = 2
NS = 16
NW = NC * NS
BPW = BATCH // NW
LANES = 16
NCHUNK = BPW // LANES
IDXCH = 128
NBLK = BPW // IDXCH
CPB = IDXCH // LANES

OFF_W1 = 0
OFF_B1 = 1024
OFF_W2 = 1056
OFF_B2 = 1088
WPACK = 1104

_SC_PARAMS = pltpu.CompilerParams(
    needs_layout_passes=False, use_tc_tiling_on_sc=False)
_MESH = dict(core_axis_name="c", subcore_axis_name="s",
             num_cores=NC, num_subcores=NS)


def _stage_ids(id_hbm, wid, idx, row):
    pltpu.sync_copy(id_hbm.at[pl.ds(pl.multiple_of(wid * BPW, 8), BPW)], idx)
    for j in range(NBLK):
        for o in range(CPB):
            v = idx[pl.ds(j * IDXCH + o * LANES, LANES)]
            row[j, pl.ds(o * LANES, LANES)] = v


def _user_body(du_hbm, ut_hbm, wp_hbm, hb_hbm,
               idx, row, buf, hbuf, wv, sem):
    c = lax.axis_index("c")
    s = lax.axis_index("s")
    wid = s * NC + c

    pltpu.sync_copy(wp_hbm, wv)
    _stage_ids(du_hbm, wid, idx, row)

    cps = [pltpu.async_copy(ut_hbm.at[row.at[j]],
                            buf.at[pl.ds(j * IDXCH, IDXCH)], sem)
           for j in range(NBLK)]

    riota = lax.iota(jnp.int32, LANES)

    def chunk(ci, carry):
        rows = ci * LANES + riota
        cols = [jnp.full((LANES,), k, jnp.int32) for k in range(EMB)]
        feats = [plsc.load_gather(buf, [rows, cols[k]]) for k in range(EMB)]
        b1a = wv[pl.ds(OFF_B1, LANES)]
        b1b = wv[pl.ds(OFF_B1 + LANES, LANES)]
        for jf in range(32):
            wa = wv[pl.ds(OFF_W1 + jf * 32, LANES)]
            bj = b1a[jf] if jf < LANES else b1b[jf - LANES]
            h = jnp.full((LANES,), 0.0, jnp.float32) + bj
            for k in range(EMB):
                h = h + feats[k] * wa[k]
            hbuf[jf, pl.ds(ci * LANES, LANES)] = h
        return carry

    for j in range(NBLK):
        cps[j].wait()
        lax.fori_loop(j * CPB, (j + 1) * CPB, chunk, 0)

    pltpu.sync_copy(hbuf, hb_hbm.at[wid])


def _item_body(di_hbm, it_hbm, wp_hbm, hb_hbm, out_hbm,
               idx, row, buf, hbuf, wv, logits_v, sem, sem2):
    c = lax.axis_index("c")
    s = lax.axis_index("s")
    wid = s * NC + c

    pltpu.sync_copy(wp_hbm, wv)
    hb_cp = pltpu.async_copy(hb_hbm.at[wid], hbuf, sem2)
    _stage_ids(di_hbm, wid, idx, row)

    cps = [pltpu.async_copy(it_hbm.at[row.at[j]],
                            buf.at[pl.ds(j * IDXCH, IDXCH)], sem)
           for j in range(NBLK)]

    riota = lax.iota(jnp.int32, LANES)

    def chunk(ci, carry):
        rows = ci * LANES + riota
        cols = [jnp.full((LANES,), k, jnp.int32) for k in range(EMB)]
        feats = [plsc.load_gather(buf, [rows, cols[k]]) for k in range(EMB)]
        w2a = wv[pl.ds(OFF_W2, LANES)]
        w2b = wv[pl.ds(OFF_W2 + LANES, LANES)]
        b2v = wv[pl.ds(OFF_B2, LANES)]
        acc2 = jnp.full((LANES,), 0.0, jnp.float32)
        for jf in range(32):
            wb = wv[pl.ds(OFF_W1 + jf * 32 + LANES, LANES)]
            h = hbuf[jf, pl.ds(ci * LANES, LANES)]
            for k in range(EMB):
                h = h + feats[k] * wb[k]
            h = jnp.maximum(h, 0.0)
            w2j = w2a[jf] if jf < LANES else w2b[jf - LANES]
            acc2 = acc2 + h * w2j
        logits_v[pl.ds(ci * LANES, LANES)] = acc2 + b2v[0]
        return carry

    hb_cp.wait()
    for j in range(NBLK):
        cps[j].wait()
        lax.fori_loop(j * CPB, (j + 1) * CPB, chunk, 0)

    pltpu.sync_copy(
        logits_v, out_hbm.at[pl.ds(pl.multiple_of(wid * BPW, 8), BPW)])


@jax.jit
def _run(du, di, ut, it, wpack):
    user_f = pl.kernel(
        _user_body,
        out_type=jax.ShapeDtypeStruct((NW, 32, BPW), jnp.float32),
        mesh=plsc.VectorSubcoreMesh(**_MESH),
        compiler_params=_SC_PARAMS,
        scratch_types=[
            pltpu.VMEM((BPW,), jnp.int32),
            pltpu.VMEM((NBLK, IDXCH), jnp.int32),
            pltpu.VMEM((BPW, EMB), jnp.float32),
            pltpu.VMEM((32, BPW), jnp.float32),
            pltpu.VMEM((WPACK,), jnp.float32),
            pltpu.SemaphoreType.DMA,
        ],
    )
    hb = user_f(du, ut, wpack)
    item_f = pl.kernel(
        _item_body,
        out_type=jax.ShapeDtypeStruct((BATCH,), jnp.float32),
        mesh=plsc.VectorSubcoreMesh(**_MESH),
        compiler_params=_SC_PARAMS,
        scratch_types=[
            pltpu.VMEM((BPW,), jnp.int32),
            pltpu.VMEM((NBLK, IDXCH), jnp.int32),
            pltpu.VMEM((BPW, EMB), jnp.float32),
            pltpu.VMEM((32, BPW), jnp.float32),
            pltpu.VMEM((WPACK,), jnp.float32),
            pltpu.VMEM((BPW,), jnp.float32),
            pltpu.SemaphoreType.DMA,
            pltpu.SemaphoreType.DMA,
        ],
    )
    return item_f(di, it, wpack, hb)


def kernel(dataUser, dataItem, user_table, item_table, W1, b1, W2, b2):
    du = dataUser.astype(jnp.int32)
    di = dataItem.astype(jnp.int32)
    wpack = jnp.concatenate([
        W1.reshape(-1), b1.reshape(-1), W2.reshape(-1), b2.reshape(-1),
        jnp.zeros((WPACK - (OFF_B2 + 1),), jnp.float32)])
    out = _run(du, di, user_table, item_table, wpack)
    return out.reshape(BATCH, 1)


# speedup vs baseline: 1.3677x; 1.1489x over previous
import jax
import jax.numpy as jnp
from jax import lax
from jax.experimental import pallas as pl
from jax.experimental.pallas import tpu as pltpu
from jax.experimental.pallas import tpu_sc as plsc

VOCAB = 100000
EMB = 16
BATCH = 16384
NC = 2
NS = 16
NW = NC * NS
BPW = BATCH // NW
LANES = 16
IDXCH = 128
NBLK = BPW // IDXCH
CPB = IDXCH // LANES

OFF_W1 = 0
OFF_B1 = 1024
OFF_W2 = 1056
OFF_B2 = 1088
WPACK = 1104

_SC_PARAMS = pltpu.CompilerParams(
    needs_layout_passes=False, use_tc_tiling_on_sc=False)
_MESH = dict(core_axis_name="c", subcore_axis_name="s",
             num_cores=NC, num_subcores=NS)


def _stage_eidx(id_hbm, wid, idx, eidx):
    pltpu.sync_copy(id_hbm.at[pl.ds(pl.multiple_of(wid * BPW, 8), BPW)], idx)
    for j in range(NBLK):
        for o in range(CPB):
            v = idx[pl.ds(j * IDXCH + o * LANES, LANES)]
            for k in range(EMB):
                eidx[pl.ds((j * EMB + k) * IDXCH + o * LANES, LANES)] = (
                    v + k * VOCAB)


def _fire_gathers(tab_flat, eidx, buf, sem):
    return [pltpu.async_copy(tab_flat.at[eidx.at[pl.ds(r * IDXCH, IDXCH)]],
                             buf.at[pl.ds(r * IDXCH, IDXCH)], sem)
            for r in range(NBLK * EMB)]


def _user_body(du_hbm, ut_flat, wp_hbm, hb_hbm, idx, eidx, buf, hbuf, wv, sem):
    c = lax.axis_index("c")
    s = lax.axis_index("s")
    wid = s * NC + c

    pltpu.sync_copy(wp_hbm, wv)
    _stage_eidx(du_hbm, wid, idx, eidx)
    cps = _fire_gathers(ut_flat, eidx, buf, sem)

    for j in range(NBLK):
        for k in range(EMB):
            cps[j * EMB + k].wait()

        def chunk(ci, carry):
            feats = [buf[pl.ds((j * EMB + k) * IDXCH + ci * LANES, LANES)]
                     for k in range(EMB)]
            b1a = wv[pl.ds(OFF_B1, LANES)]
            b1b = wv[pl.ds(OFF_B1 + LANES, LANES)]
            for jf in range(32):
                wa = wv[pl.ds(OFF_W1 + jf * 32, LANES)]
                bj = b1a[jf] if jf < LANES else b1b[jf - LANES]
                h = jnp.full((LANES,), 0.0, jnp.float32) + bj
                for k in range(EMB):
                    h = h + feats[k] * wa[k]
                hbuf[jf, pl.ds((j * CPB + ci) * LANES, LANES)] = h
            return carry

        lax.fori_loop(0, CPB, chunk, 0)

    pltpu.sync_copy(hbuf, hb_hbm.at[wid])


def _item_body(di_hbm, it_flat, wp_hbm, hb_hbm, out_hbm,
               idx, eidx, buf, hbuf, wv, logits_v, sem, sem2):
    c = lax.axis_index("c")
    s = lax.axis_index("s")
    wid = s * NC + c

    pltpu.sync_copy(wp_hbm, wv)
    hb_cp = pltpu.async_copy(hb_hbm.at[wid], hbuf, sem2)
    _stage_eidx(di_hbm, wid, idx, eidx)
    cps = _fire_gathers(it_flat, eidx, buf, sem)

    hb_cp.wait()
    for j in range(NBLK):
        for k in range(EMB):
            cps[j * EMB + k].wait()

        def chunk(ci, carry):
            feats = [buf[pl.ds((j * EMB + k) * IDXCH + ci * LANES, LANES)]
                     for k in range(EMB)]
            w2a = wv[pl.ds(OFF_W2, LANES)]
            w2b = wv[pl.ds(OFF_W2 + LANES, LANES)]
            b2v = wv[pl.ds(OFF_B2, LANES)]
            acc2 = jnp.full((LANES,), 0.0, jnp.float32)
            for jf in range(32):
                wb = wv[pl.ds(OFF_W1 + jf * 32 + LANES, LANES)]
                h = hbuf[jf, pl.ds((j * CPB + ci) * LANES, LANES)]
                for k in range(EMB):
                    h = h + feats[k] * wb[k]
                h = jnp.maximum(h, 0.0)
                w2j = w2a[jf] if jf < LANES else w2b[jf - LANES]
                acc2 = acc2 + h * w2j
            logits_v[pl.ds((j * CPB + ci) * LANES, LANES)] = acc2 + b2v[0]
            return carry

        lax.fori_loop(0, CPB, chunk, 0)

    pltpu.sync_copy(
        logits_v, out_hbm.at[pl.ds(pl.multiple_of(wid * BPW, 8), BPW)])


@jax.jit
def _run(du, di, utf, itf, wpack):
    user_f = pl.kernel(
        _user_body,
        out_type=jax.ShapeDtypeStruct((NW, 32, BPW), jnp.float32),
        mesh=plsc.VectorSubcoreMesh(**_MESH),
        compiler_params=_SC_PARAMS,
        scratch_types=[
            pltpu.VMEM((BPW,), jnp.int32),
            pltpu.VMEM((NBLK * EMB * IDXCH,), jnp.int32),
            pltpu.VMEM((NBLK * EMB * IDXCH,), jnp.float32),
            pltpu.VMEM((32, BPW), jnp.float32),
            pltpu.VMEM((WPACK,), jnp.float32),
            pltpu.SemaphoreType.DMA,
        ],
    )
    hb = user_f(du, utf, wpack)
    item_f = pl.kernel(
        _item_body,
        out_type=jax.ShapeDtypeStruct((BATCH,), jnp.float32),
        mesh=plsc.VectorSubcoreMesh(**_MESH),
        compiler_params=_SC_PARAMS,
        scratch_types=[
            pltpu.VMEM((BPW,), jnp.int32),
            pltpu.VMEM((NBLK * EMB * IDXCH,), jnp.int32),
            pltpu.VMEM((NBLK * EMB * IDXCH,), jnp.float32),
            pltpu.VMEM((32, BPW), jnp.float32),
            pltpu.VMEM((WPACK,), jnp.float32),
            pltpu.VMEM((BPW,), jnp.float32),
            pltpu.SemaphoreType.DMA,
            pltpu.SemaphoreType.DMA,
        ],
    )
    return item_f(di, itf, wpack, hb)


def kernel(dataUser, dataItem, user_table, item_table, W1, b1, W2, b2):
    du = dataUser.astype(jnp.int32)
    di = dataItem.astype(jnp.int32)
    utf = user_table.T.reshape(-1)
    itf = item_table.T.reshape(-1)
    wpack = jnp.concatenate([
        W1.reshape(-1), b1.reshape(-1), W2.reshape(-1), b2.reshape(-1),
        jnp.zeros((WPACK - (OFF_B2 + 1),), jnp.float32)])
    out = _run(du, di, utf, itf, wpack)
    return out.reshape(BATCH, 1)
